# baseline (speedup 1.0000x reference)
import functools

import jax
import jax.numpy as jnp
from jax import lax
from jax.experimental import pallas as pl
from jax.experimental.pallas import tpu as pltpu
from jax.experimental.pallas import tpu_sc as plsc

L_SPINS = 20
BATCH = 16384
DELTA = 1e-15
LN2 = 0.6931471805599453
PI = 3.141592653589793

NC = 2
NS = 16
LANES = 16
NW = NC * NS
B_PER_W = BATCH // NW
CHUNK = 128
NCHUNK = B_PER_W // CHUNK


def _log_angle(k):
    a = jnp.abs(k + jnp.float32(DELTA))
    bits = lax.bitcast_convert_type(a, jnp.int32)
    e = ((bits >> 23) & 0xFF).astype(jnp.float32) - 127.0
    m = lax.bitcast_convert_type((bits & 0x7FFFFF) | 0x3F800000, jnp.float32)
    t = (m - 1.0) / (m + 1.0)
    t2 = t * t
    poly = t * (2.0 + t2 * (2.0 / 3.0 + t2 * (2.0 / 5.0 + t2 * (2.0 / 7.0 + t2 * (2.0 / 9.0)))))
    re = e * jnp.float32(LN2) + poly
    im = jnp.where(k < 0.0, jnp.float32(PI), jnp.float32(0.0))
    return re, im


def _idx_body(s_ref, idx_ref):
    w = jnp.int32(1) << (jnp.int32(L_SPINS - 1)
                         - lax.broadcasted_iota(jnp.int32, (L_SPINS, 1), 0))
    idx_ref[...] = jnp.sum(s_ref[...] * w, axis=0, keepdims=True)


_tc_indices = pl.pallas_call(
    _idx_body,
    grid=(1,),
    in_specs=[pl.BlockSpec((L_SPINS, BATCH), lambda i: (0, 0))],
    out_specs=pl.BlockSpec((1, BATCH), lambda i: (0, 0)),
    out_shape=jax.ShapeDtypeStruct((1, BATCH), jnp.int32),
)


@functools.partial(
    pl.kernel,
    mesh=plsc.VectorSubcoreMesh(core_axis_name="c", subcore_axis_name="s"),
    out_type=jax.ShapeDtypeStruct((2, BATCH), jnp.float32),
    scratch_types=[
        pltpu.VMEM((NCHUNK, CHUNK), jnp.int32),
        pltpu.VMEM((B_PER_W,), jnp.float32),
        pltpu.VMEM((2, B_PER_W), jnp.float32),
        pltpu.SemaphoreType.DMA,
        pltpu.SemaphoreType.DMA,
    ],
)
def _sc_lookup(idx_hbm, table_hbm, out_hbm,
               idx_v, k_v, ri_v, gsem, osem):
    wid = lax.axis_index("s") * NC + lax.axis_index("c")
    base = wid * B_PER_W

    idx_loads = [
        pltpu.async_copy(idx_hbm.at[0, pl.ds(base + c * CHUNK, CHUNK)],
                         idx_v.at[c], osem)
        for c in range(NCHUNK)
    ]
    gathers = []
    for c in range(NCHUNK):
        idx_loads[c].wait()
        gathers.append(pltpu.async_copy(table_hbm.at[idx_v.at[c]],
                                        k_v.at[pl.ds(c * CHUNK, CHUNK)], gsem))

    for g in gathers:
        g.wait()

    def epilogue(j, carry):
        sl = pl.ds(j * LANES, LANES)
        re, im = _log_angle(k_v[sl])
        ri_v[0, sl] = re
        ri_v[1, sl] = im
        return carry
    lax.fori_loop(0, B_PER_W // LANES, epilogue, 0)

    pltpu.async_copy(ri_v, out_hbm.at[:, pl.ds(base, B_PER_W)], osem).wait()


def kernel(s, kernel):
    idx = _tc_indices(s)
    ri = _sc_lookup(idx, kernel)
    return lax.complex(ri[0], ri[1])


# speedup vs baseline: 1.0036x; 1.0036x over previous
import functools

import jax
import jax.numpy as jnp
from jax import lax
from jax.experimental import pallas as pl
from jax.experimental.pallas import tpu as pltpu
from jax.experimental.pallas import tpu_sc as plsc

L_SPINS = 20
BATCH = 16384
DELTA = 1e-15
LN2 = 0.6931471805599453
PI = 3.141592653589793

NC = 2
NS = 16
LANES = 16
NW = NC * NS
B_PER_W = BATCH // NW
CHUNK = 128
NCHUNK = B_PER_W // CHUNK


def _log_angle(k):
    a = jnp.abs(k + jnp.float32(DELTA))
    bits = lax.bitcast_convert_type(a, jnp.int32)
    e = ((bits >> 23) & 0xFF).astype(jnp.float32) - 127.0
    m = lax.bitcast_convert_type((bits & 0x7FFFFF) | 0x3F800000, jnp.float32)
    t = (m - 1.0) / (m + 1.0)
    t2 = t * t
    poly = t * (2.0 + t2 * (2.0 / 3.0 + t2 * (2.0 / 5.0 + t2 * (2.0 / 7.0 + t2 * (2.0 / 9.0)))))
    re = e * jnp.float32(LN2) + poly
    im = jnp.where(k < 0.0, jnp.float32(PI), jnp.float32(0.0))
    return re, im


def _idx_body(s_ref, idx_ref):
    sh = (jnp.int32(L_SPINS - 1)
          - lax.broadcasted_iota(jnp.int32, (L_SPINS, 1), 0))
    idx_ref[...] = jnp.sum(s_ref[...] << sh, axis=0, keepdims=True)


_tc_indices = pl.pallas_call(
    _idx_body,
    out_shape=jax.ShapeDtypeStruct((1, BATCH), jnp.int32),
)


@functools.partial(
    pl.kernel,
    mesh=plsc.VectorSubcoreMesh(core_axis_name="c", subcore_axis_name="s"),
    out_type=jax.ShapeDtypeStruct((2, BATCH), jnp.float32),
    scratch_types=[
        pltpu.VMEM((B_PER_W,), jnp.int32),
        pltpu.VMEM((B_PER_W,), jnp.float32),
        pltpu.VMEM((2, B_PER_W), jnp.float32),
        pltpu.SemaphoreType.DMA,
        pltpu.SemaphoreType.DMA,
    ],
)
def _sc_lookup(idx_hbm, table_hbm, out_hbm,
               idx_v, k_v, ri_v, gsem, osem):
    wid = lax.axis_index("s") * NC + lax.axis_index("c")
    base = wid * B_PER_W

    pltpu.sync_copy(idx_hbm.at[0, pl.ds(base, B_PER_W)], idx_v)
    gathers = [
        pltpu.async_copy(table_hbm.at[idx_v.at[pl.ds(c * CHUNK, CHUNK)]],
                         k_v.at[pl.ds(c * CHUNK, CHUNK)], gsem)
        for c in range(NCHUNK)
    ]

    for g in gathers:
        g.wait()

    def epilogue(j, carry):
        sl = pl.ds(j * LANES, LANES)
        re, im = _log_angle(k_v[sl])
        ri_v[0, sl] = re
        ri_v[1, sl] = im
        return carry
    lax.fori_loop(0, B_PER_W // LANES, epilogue, 0)

    pltpu.async_copy(ri_v, out_hbm.at[:, pl.ds(base, B_PER_W)], osem).wait()


def kernel(s, kernel):
    idx = _tc_indices(s)
    ri = _sc_lookup(idx, kernel)
    return lax.complex(ri[0], ri[1])
